# baseline (speedup 1.0000x reference)
import functools

import jax
import jax.numpy as jnp
from jax import lax
from jax.experimental import pallas as pl
from jax.experimental.pallas import tpu as pltpu
from jax.experimental.pallas import tpu_sc as plsc

N = 16384
C = 1000
L = 16
NC, NS = 2, 16
NW = NC * NS
BPW = N // NW
NCH = BPW // L
NB = 8
BCAP = 640
MAXCH = NCH + NB

_LN2 = 0.6931471805599453
_SQRT2 = 1.4142135623730951


def _vlog(x):
    bits = lax.bitcast_convert_type(x, jnp.int32)
    e = lax.shift_right_logical(bits, 23) - 127
    m = lax.bitcast_convert_type((bits & 0x007FFFFF) | 0x3F800000, jnp.float32)
    big = m > _SQRT2
    m = jnp.where(big, m * 0.5, m)
    e = e + jnp.where(big, 1, 0)
    s = (m - 1.0) / (m + 1.0)
    z = s * s
    p = 1.0 + z * (1 / 3 + z * (1 / 5 + z * (1 / 7 + z * (1 / 9))))
    return e.astype(jnp.float32) * _LN2 + 2.0 * s * p


_MESH = plsc.VectorSubcoreMesh(core_axis_name="c", subcore_axis_name="s")


@functools.partial(
    pl.kernel,
    mesh=_MESH,
    out_type=jax.ShapeDtypeStruct((NW, L), jnp.float32),
    compiler_params=pltpu.CompilerParams(needs_layout_passes=False,
                                         use_tc_tiling_on_sc=True),
    scratch_types=[
        pltpu.VMEM((BPW,), jnp.int32),
        pltpu.VMEM((NB, BCAP), jnp.int32),
        pltpu.VMEM((NB, BCAP), jnp.int32),
        pltpu.VMEM((MAXCH * L, 128), jnp.float32),
        pltpu.VMEM((L,), jnp.float32),
        pltpu.SemaphoreType.DMA,
    ],
)
def _nll_partials(table_hbm, tgt_hbm, out_hbm, tgt_v, rows_b, cols_b, win_v,
                  acc_v, sem):
    wid = lax.axis_index("s") * NC + lax.axis_index("c")
    base = wid * BPW
    pltpu.sync_copy(tgt_hbm.at[pl.ds(base, BPW)], tgt_v)
    lane = lax.iota(jnp.int32, L)

    offs = [jnp.int32(0)] * NB
    for j in range(NCH):
        t = tgt_v[pl.ds(j * L, L)]
        rows = base + j * L + lane
        kvec = lax.shift_right_logical(t, 7)
        col = t & 127
        for k in range(NB):
            msk = kvec == k
            plsc.store_compressed(rows_b.at[k, pl.ds(offs[k], L)], rows, mask=msk)
            plsc.store_compressed(cols_b.at[k, pl.ds(offs[k], L)], col, mask=msk)
            offs[k] = offs[k] + jnp.sum(msk.astype(jnp.int32))

    for k in range(NB):
        rows_b[k, pl.ds(offs[k], L)] = base + lane
        cols_b[k, pl.ds(offs[k], L)] = lane

    nchunks = [lax.shift_right_logical(offs[k] + (L - 1), 4) for k in range(NB)]
    gptr = jnp.int32(0)
    for k in range(NB):
        def _fire(cc, gp, k=k):
            start = wid * 0 + k * 128
            pltpu.async_copy(
                table_hbm.at[rows_b.at[k, pl.ds(cc * L, L)],
                             pl.ds(start, 128)],
                win_v.at[pl.ds(gp * L, L)], sem)
            return gp + 1
        gptr = lax.fori_loop(0, nchunks[k], _fire, gptr)

    def _drain(cc, carry):
        pltpu.make_async_copy(
            table_hbm.at[pl.ds(0, L), pl.ds(0, 128)],
            win_v.at[pl.ds(0, L)], sem).wait()
        return carry
    lax.fori_loop(0, gptr, _drain, jnp.int32(0))

    acc = jnp.zeros((L,), jnp.float32)
    gptr = jnp.int32(0)
    for k in range(NB):
        def _pick(cc, carry, k=k):
            a, gp = carry
            cols = cols_b[k, pl.ds(cc * L, L)]
            vals = plsc.load_gather(win_v, [gp * L + lane, cols])
            valid = (cc * L + lane) < offs[k]
            a = a + jnp.where(valid, _vlog(vals), 0.0)
            return a, gp + 1
        acc, gptr = lax.fori_loop(0, nchunks[k], _pick, (acc, gptr))

    acc_v[...] = acc
    pltpu.sync_copy(acc_v, out_hbm.at[wid])


def kernel(input, target):
    partials = _nll_partials(input, target.astype(jnp.int32))
    return -jnp.sum(partials) / jnp.float32(N)


# speedup vs baseline: 2.8894x; 2.8894x over previous
import functools

import jax
import jax.numpy as jnp
from jax import lax
from jax.experimental import pallas as pl
from jax.experimental.pallas import tpu as pltpu
from jax.experimental.pallas import tpu_sc as plsc

N = 16384
C = 1000
L = 16
NC, NS = 2, 16
NW = NC * NS
BPW = N // NW
WIN = 128
NWIN = BPW // WIN

_LN2 = 0.6931471805599453
_SQRT2 = 1.4142135623730951


def _vlog(x):
    bits = lax.bitcast_convert_type(x, jnp.int32)
    e = lax.shift_right_logical(bits, 23) - 127
    m = lax.bitcast_convert_type((bits & 0x007FFFFF) | 0x3F800000, jnp.float32)
    big = m > _SQRT2
    m = jnp.where(big, m * 0.5, m)
    e = e + jnp.where(big, 1, 0)
    s = (m - 1.0) / (m + 1.0)
    z = s * s
    p = 1.0 + z * (1 / 3 + z * (1 / 5 + z * (1 / 7 + z * (1 / 9))))
    return e.astype(jnp.float32) * _LN2 + 2.0 * s * p


_MESH = plsc.VectorSubcoreMesh(core_axis_name="c", subcore_axis_name="s")


@functools.partial(
    pl.kernel,
    mesh=_MESH,
    out_type=jax.ShapeDtypeStruct((NW, L), jnp.float32),
    compiler_params=pltpu.CompilerParams(needs_layout_passes=False),
    scratch_types=[
        pltpu.VMEM((BPW,), jnp.int32),
        pltpu.VMEM((NWIN, WIN, WIN), jnp.float32),
        pltpu.VMEM((L,), jnp.float32),
        pltpu.SemaphoreType.DMA,
    ],
)
def _nll_partials(tableT_hbm, tgt_hbm, out_hbm, tgt_v, win_v, acc_v, sem):
    wid = lax.axis_index("s") * NC + lax.axis_index("c")
    base = wid * BPW
    pltpu.sync_copy(tgt_hbm.at[pl.ds(base, BPW)], tgt_v)
    lane = lax.iota(jnp.int32, L)

    copies = []
    for w in range(NWIN):
        copies.append(pltpu.async_copy(
            tableT_hbm.at[tgt_v.at[pl.ds(w * WIN, WIN)],
                          pl.ds(base + w * WIN, WIN)],
            win_v.at[w], sem))
    for cp in copies:
        cp.wait()

    acc = jnp.zeros((L,), jnp.float32)
    for w in range(NWIN):
        for sub in range(WIN // L):
            d = sub * L + lane
            vals = plsc.load_gather(win_v.at[w], [d, d])
            acc = acc + _vlog(vals)

    acc_v[...] = acc
    pltpu.sync_copy(acc_v, out_hbm.at[wid])


def kernel(input, target):
    partials = _nll_partials(input.T, target.astype(jnp.int32))
    return -jnp.sum(partials) / jnp.float32(N)


# speedup vs baseline: 3.1641x; 1.0951x over previous
import functools

import jax
import jax.numpy as jnp
from jax import lax
from jax.experimental import pallas as pl
from jax.experimental.pallas import tpu as pltpu
from jax.experimental.pallas import tpu_sc as plsc

N = 16384
C = 1000
L = 16
NC, NS = 2, 16
NW = NC * NS
BPW = N // NW
WIN = 128
NWIN = BPW // WIN

_LN2 = 0.6931471805599453
_P = (-1.9316664196629012, 3.4982118829630044, -2.4207929905996237,
      1.1047965807705125, -0.2806291682866353, 0.030102247599643327)


def _vlog(x):
    bits = lax.bitcast_convert_type(x, jnp.int32)
    e = lax.shift_right_logical(bits, 23) - 127
    m = lax.bitcast_convert_type((bits & 0x007FFFFF) | 0x3F800000, jnp.float32)
    p = _P[5]
    for c in (_P[4], _P[3], _P[2], _P[1], _P[0]):
        p = p * m + c
    return e.astype(jnp.float32) * _LN2 + p


_MESH = plsc.VectorSubcoreMesh(core_axis_name="c", subcore_axis_name="s")


@functools.partial(
    pl.kernel,
    mesh=_MESH,
    out_type=jax.ShapeDtypeStruct((NW, L), jnp.float32),
    compiler_params=pltpu.CompilerParams(needs_layout_passes=False),
    scratch_types=[
        pltpu.VMEM((BPW,), jnp.int32),
        pltpu.VMEM((NWIN, WIN, WIN), jnp.float32),
        pltpu.VMEM((L,), jnp.float32),
        pltpu.SemaphoreType.DMA,
    ],
)
def _nll_partials(tableT_hbm, tgt_hbm, out_hbm, tgt_v, win_v, acc_v, sem):
    wid = lax.axis_index("s") * NC + lax.axis_index("c")
    base = wid * BPW
    pltpu.sync_copy(tgt_hbm.at[pl.ds(base, BPW)], tgt_v)
    lane = lax.iota(jnp.int32, L)

    copies = [pltpu.async_copy(
        tableT_hbm.at[tgt_v.at[pl.ds(w * WIN, WIN)],
                      pl.ds(base + w * WIN, WIN)],
        win_v.at[w], sem) for w in range(NWIN)]

    acc = jnp.zeros((L,), jnp.float32)
    for w in range(NWIN):
        copies[w].wait()

        def _chunk(j, a, w=w):
            d = j * L + lane
            return a + _vlog(plsc.load_gather(win_v.at[w], [d, d]))
        acc = lax.fori_loop(0, WIN // L, _chunk, acc)

    acc_v[...] = acc * jnp.float32(-1.0 / N)
    pltpu.sync_copy(acc_v, out_hbm.at[wid])


def kernel(input, target):
    partials = _nll_partials(input.T, target.astype(jnp.int32))
    return jnp.sum(partials)


# speedup vs baseline: 3.1667x; 1.0008x over previous
import functools

import jax
import jax.numpy as jnp
from jax import lax
from jax.experimental import pallas as pl
from jax.experimental.pallas import tpu as pltpu
from jax.experimental.pallas import tpu_sc as plsc

N = 16384
C = 1000
L = 16
NC, NS = 2, 16
NW = NC * NS
BPW = N // NW
WIN = 128
NWIN = BPW // WIN

_LN2 = 0.6931471805599453
_P = (-1.9316664196629012, 3.4982118829630044, -2.4207929905996237,
      1.1047965807705125, -0.2806291682866353, 0.030102247599643327)


def _vlog(x):
    bits = lax.bitcast_convert_type(x, jnp.int32)
    e = lax.shift_right_logical(bits, 23) - 127
    m = lax.bitcast_convert_type((bits & 0x007FFFFF) | 0x3F800000, jnp.float32)
    p = _P[5]
    for c in (_P[4], _P[3], _P[2], _P[1], _P[0]):
        p = p * m + c
    return e.astype(jnp.float32) * _LN2 + p


_MESH = plsc.VectorSubcoreMesh(core_axis_name="c", subcore_axis_name="s")


@functools.partial(
    pl.kernel,
    mesh=_MESH,
    out_type=jax.ShapeDtypeStruct((NW, L), jnp.float32),
    compiler_params=pltpu.CompilerParams(needs_layout_passes=False,
                                         skip_device_barrier=True),
    scratch_types=[
        pltpu.VMEM((BPW,), jnp.int32),
        pltpu.VMEM((NWIN, WIN, WIN), jnp.float32),
        pltpu.VMEM((L,), jnp.float32),
        pltpu.SemaphoreType.DMA,
    ],
)
def _nll_partials(tableT_hbm, tgt_hbm, out_hbm, tgt_v, win_v, acc_v, sem):
    wid = lax.axis_index("s") * NC + lax.axis_index("c")
    base = wid * BPW
    pltpu.sync_copy(tgt_hbm.at[pl.ds(base, BPW)], tgt_v)
    lane = lax.iota(jnp.int32, L)

    copies = [pltpu.async_copy(
        tableT_hbm.at[tgt_v.at[pl.ds(w * WIN, WIN)],
                      pl.ds(base + w * WIN, WIN)],
        win_v.at[w], sem) for w in range(NWIN)]

    acc = jnp.zeros((L,), jnp.float32)
    for w in range(NWIN):
        copies[w].wait()

        def _chunk(j, a, w=w):
            d = j * L + lane
            return a + _vlog(plsc.load_gather(win_v.at[w], [d, d]))
        acc = lax.fori_loop(0, WIN // L, _chunk, acc)

    acc_v[...] = acc * jnp.float32(-1.0 / N)
    pltpu.sync_copy(acc_v, out_hbm.at[wid])


def kernel(input, target):
    partials = _nll_partials(input.T, target.astype(jnp.int32))
    return jnp.sum(partials)


# speedup vs baseline: 3.4724x; 1.0965x over previous
import functools

import jax
import jax.numpy as jnp
from jax import lax
from jax.experimental import pallas as pl
from jax.experimental.pallas import tpu as pltpu
from jax.experimental.pallas import tpu_sc as plsc

N = 16384
C = 1000
L = 16
NC, NS = 2, 16
NW = NC * NS
BPW = N // NW
WIN = 128
NWIN = BPW // WIN

_LN2 = 0.6931471805599453
_P = (-1.9316664196629012, 3.4982118829630044, -2.4207929905996237,
      1.1047965807705125, -0.2806291682866353, 0.030102247599643327)


def _vlog(x):
    bits = lax.bitcast_convert_type(x, jnp.int32)
    e = lax.shift_right_logical(bits, 23) - 127
    m = lax.bitcast_convert_type((bits & 0x007FFFFF) | 0x3F800000, jnp.float32)
    p = _P[5]
    for c in (_P[4], _P[3], _P[2], _P[1], _P[0]):
        p = p * m + c
    return e.astype(jnp.float32) * _LN2 + p


_MESH = plsc.VectorSubcoreMesh(core_axis_name="c", subcore_axis_name="s")


@functools.partial(
    pl.kernel,
    mesh=_MESH,
    out_type=jax.ShapeDtypeStruct((NW, L), jnp.float32),
    compiler_params=pltpu.CompilerParams(needs_layout_passes=False,
                                         skip_device_barrier=True),
    scratch_types=[
        pltpu.VMEM((BPW,), jnp.int32),
        pltpu.VMEM((NWIN, WIN, WIN), jnp.float32),
        pltpu.VMEM((L,), jnp.float32),
        pltpu.SemaphoreType.DMA,
    ],
)
def _nll_partials(tableT_hbm, tgt_hbm, out_hbm, tgt_v, win_v, acc_v, sem):
    wid = lax.axis_index("s") * NC + lax.axis_index("c")
    base = wid * BPW
    pltpu.sync_copy(tgt_hbm.at[pl.ds(base, BPW)], tgt_v)
    lane = lax.iota(jnp.int32, L)

    def _fire(w, carry):
        pltpu.async_copy(
            tableT_hbm.at[tgt_v.at[pl.ds(w * WIN, WIN)],
                          pl.ds(base + w * WIN, WIN)],
            win_v.at[w], sem)
        return carry
    lax.fori_loop(0, NWIN, _fire, jnp.int32(0))

    def _window(w, a):
        pltpu.make_async_copy(
            tableT_hbm.at[tgt_v.at[pl.ds(w * WIN, WIN)],
                          pl.ds(base + w * WIN, WIN)],
            win_v.at[w], sem).wait()

        def _chunk(j, aa):
            d = j * L + lane
            return aa + _vlog(plsc.load_gather(win_v.at[w], [d, d]))
        return lax.fori_loop(0, WIN // L, _chunk, a)
    acc = lax.fori_loop(0, NWIN, _window, jnp.zeros((L,), jnp.float32))

    acc_v[...] = acc * jnp.float32(-1.0 / N)
    pltpu.sync_copy(acc_v, out_hbm.at[wid])


def kernel(input, target):
    partials = _nll_partials(input.T, target.astype(jnp.int32))
    return jnp.sum(partials)


# speedup vs baseline: 3.4976x; 1.0073x over previous
import functools

import jax
import jax.numpy as jnp
from jax import lax
from jax.experimental import pallas as pl
from jax.experimental.pallas import tpu as pltpu
from jax.experimental.pallas import tpu_sc as plsc

N = 16384
C = 1000
L = 16
NC, NS = 2, 16
NW = NC * NS
BPW = N // NW
WIN = 128
NWIN = BPW // WIN

_LN2 = 0.6931471805599453
_P = (-1.9316664196629012, 3.4982118829630044, -2.4207929905996237,
      1.1047965807705125, -0.2806291682866353, 0.030102247599643327)


def _vlog(x):
    bits = lax.bitcast_convert_type(x, jnp.int32)
    e = lax.shift_right_logical(bits, 23) - 127
    m = lax.bitcast_convert_type((bits & 0x007FFFFF) | 0x3F800000, jnp.float32)
    p = _P[5]
    for c in (_P[4], _P[3], _P[2], _P[1], _P[0]):
        p = p * m + c
    return e.astype(jnp.float32) * _LN2 + p


_MESH = plsc.VectorSubcoreMesh(core_axis_name="c", subcore_axis_name="s")


@functools.partial(
    pl.kernel,
    mesh=_MESH,
    out_type=jax.ShapeDtypeStruct((NW, L), jnp.float32),
    compiler_params=pltpu.CompilerParams(needs_layout_passes=False,
                                         skip_device_barrier=True),
    scratch_types=[
        pltpu.VMEM((BPW,), jnp.int32),
        pltpu.VMEM((NWIN, WIN, WIN), jnp.float32),
        pltpu.VMEM((L,), jnp.float32),
        pltpu.SemaphoreType.DMA((NWIN,)),
    ],
)
def _nll_partials(tableT_hbm, tgt_hbm, out_hbm, tgt_v, win_v, acc_v, sem):
    wid = lax.axis_index("s") * NC + lax.axis_index("c")
    base = wid * BPW
    pltpu.sync_copy(tgt_hbm.at[pl.ds(base, BPW)], tgt_v)
    lane = lax.iota(jnp.int32, L)

    def _fire(w, carry):
        pltpu.async_copy(
            tableT_hbm.at[tgt_v.at[pl.ds(w * WIN, WIN)],
                          pl.ds(base + w * WIN, WIN)],
            win_v.at[w], sem.at[w])
        return carry
    lax.fori_loop(0, NWIN, _fire, jnp.int32(0))

    def _window(w, a):
        pltpu.make_async_copy(
            tableT_hbm.at[tgt_v.at[pl.ds(w * WIN, WIN)],
                          pl.ds(base + w * WIN, WIN)],
            win_v.at[w], sem.at[w]).wait()

        def _chunk(j, aa):
            d = j * L + lane
            return aa + _vlog(plsc.load_gather(win_v.at[w], [d, d]))
        return lax.fori_loop(0, WIN // L, _chunk, a)
    acc = lax.fori_loop(0, NWIN, _window, jnp.zeros((L,), jnp.float32))

    acc_v[...] = acc * jnp.float32(-1.0 / N)
    pltpu.sync_copy(acc_v, out_hbm.at[wid])


def kernel(input, target):
    partials = _nll_partials(input.T, target.astype(jnp.int32))
    return jnp.sum(partials)


# speedup vs baseline: 3.5209x; 1.0066x over previous
import functools

import jax
import jax.numpy as jnp
from jax import lax
from jax.experimental import pallas as pl
from jax.experimental.pallas import tpu as pltpu
from jax.experimental.pallas import tpu_sc as plsc

N = 16384
C = 1000
L = 16
NC, NS = 2, 16
NW = NC * NS
BPW = N // NW
WIN = 128
IPT = 64
NT = BPW // IPT

_LN2 = 0.6931471805599453
_P = (-1.9316664196629012, 3.4982118829630044, -2.4207929905996237,
      1.1047965807705125, -0.2806291682866353, 0.030102247599643327)


def _vlog(x):
    bits = lax.bitcast_convert_type(x, jnp.int32)
    e = lax.shift_right_logical(bits, 23) - 127
    m = lax.bitcast_convert_type((bits & 0x007FFFFF) | 0x3F800000, jnp.float32)
    p = _P[5]
    for c in (_P[4], _P[3], _P[2], _P[1], _P[0]):
        p = p * m + c
    return e.astype(jnp.float32) * _LN2 + p


_MESH = plsc.VectorSubcoreMesh(core_axis_name="c", subcore_axis_name="s")


@functools.partial(
    pl.kernel,
    mesh=_MESH,
    out_type=jax.ShapeDtypeStruct((NW, L), jnp.float32),
    compiler_params=pltpu.CompilerParams(needs_layout_passes=False,
                                         skip_device_barrier=True),
    scratch_types=[
        pltpu.VMEM((BPW,), jnp.int32),
        pltpu.VMEM((NT, IPT, WIN), jnp.float32),
        pltpu.VMEM((L,), jnp.float32),
        pltpu.SemaphoreType.DMA((NT,)),
    ],
)
def _nll_partials(tableT_hbm, tgt_hbm, out_hbm, tgt_v, win_v, acc_v, sem):
    wid = lax.axis_index("s") * NC + lax.axis_index("c")
    base = wid * BPW
    pltpu.sync_copy(tgt_hbm.at[pl.ds(base, BPW)], tgt_v)
    lane = lax.iota(jnp.int32, L)

    def _fire(w, carry):
        pltpu.async_copy(
            tableT_hbm.at[tgt_v.at[pl.ds(w * IPT, IPT)],
                          pl.ds(base + (w // 2) * WIN, WIN)],
            win_v.at[w], sem.at[w])
        return carry
    lax.fori_loop(0, NT, _fire, jnp.int32(0))

    def _window(w, a):
        pltpu.make_async_copy(
            tableT_hbm.at[tgt_v.at[pl.ds(w * IPT, IPT)],
                          pl.ds(base + (w // 2) * WIN, WIN)],
            win_v.at[w], sem.at[w]).wait()
        cbase = (w % 2) * IPT

        def _chunk(j, aa):
            d = j * L + lane
            return aa + _vlog(plsc.load_gather(win_v.at[w], [d, cbase + d]))
        return lax.fori_loop(0, IPT // L, _chunk, a)
    acc = lax.fori_loop(0, NT, _window, jnp.zeros((L,), jnp.float32))

    acc_v[...] = acc * jnp.float32(-1.0 / N)
    pltpu.sync_copy(acc_v, out_hbm.at[wid])


def kernel(input, target):
    partials = _nll_partials(input.T, target.astype(jnp.int32))
    return jnp.sum(partials)


# speedup vs baseline: 3.5413x; 1.0058x over previous
import functools

import jax
import jax.numpy as jnp
from jax import lax
from jax.experimental import pallas as pl
from jax.experimental.pallas import tpu as pltpu
from jax.experimental.pallas import tpu_sc as plsc

N = 16384
C = 1000
L = 16
NC, NS = 2, 16
NW = NC * NS
BPW = N // NW
WIN = 128
IPT = 32
NT = BPW // IPT

_LN2 = 0.6931471805599453
_P = (-1.9316664196629012, 3.4982118829630044, -2.4207929905996237,
      1.1047965807705125, -0.2806291682866353, 0.030102247599643327)


def _vlog(x):
    bits = lax.bitcast_convert_type(x, jnp.int32)
    e = lax.shift_right_logical(bits, 23) - 127
    m = lax.bitcast_convert_type((bits & 0x007FFFFF) | 0x3F800000, jnp.float32)
    p = _P[5]
    for c in (_P[4], _P[3], _P[2], _P[1], _P[0]):
        p = p * m + c
    return e.astype(jnp.float32) * _LN2 + p


_MESH = plsc.VectorSubcoreMesh(core_axis_name="c", subcore_axis_name="s")


@functools.partial(
    pl.kernel,
    mesh=_MESH,
    out_type=jax.ShapeDtypeStruct((NW, L), jnp.float32),
    compiler_params=pltpu.CompilerParams(needs_layout_passes=False,
                                         skip_device_barrier=True),
    scratch_types=[
        pltpu.VMEM((BPW,), jnp.int32),
        pltpu.VMEM((NT, IPT, WIN), jnp.float32),
        pltpu.VMEM((L,), jnp.float32),
        pltpu.SemaphoreType.DMA((NT,)),
    ],
)
def _nll_partials(tableT_hbm, tgt_hbm, out_hbm, tgt_v, win_v, acc_v, sem):
    wid = lax.axis_index("s") * NC + lax.axis_index("c")
    base = wid * BPW
    pltpu.sync_copy(tgt_hbm.at[pl.ds(base, BPW)], tgt_v)
    lane = lax.iota(jnp.int32, L)

    def _fire(w, carry):
        pltpu.async_copy(
            tableT_hbm.at[tgt_v.at[pl.ds(w * IPT, IPT)],
                          pl.ds(base + (w // 4) * WIN, WIN)],
            win_v.at[w], sem.at[w])
        return carry
    lax.fori_loop(0, NT, _fire, jnp.int32(0))

    def _window(w, a):
        pltpu.make_async_copy(
            tableT_hbm.at[tgt_v.at[pl.ds(w * IPT, IPT)],
                          pl.ds(base + (w // 4) * WIN, WIN)],
            win_v.at[w], sem.at[w]).wait()
        cbase = (w % 4) * IPT

        def _chunk(j, aa):
            d = j * L + lane
            return aa + _vlog(plsc.load_gather(win_v.at[w], [d, cbase + d]))
        return lax.fori_loop(0, IPT // L, _chunk, a)
    acc = lax.fori_loop(0, NT, _window, jnp.zeros((L,), jnp.float32))

    acc_v[...] = acc * jnp.float32(-1.0 / N)
    pltpu.sync_copy(acc_v, out_hbm.at[wid])


def kernel(input, target):
    partials = _nll_partials(input.T, target.astype(jnp.int32))
    return jnp.sum(partials)
